# baseline (speedup 1.0000x reference)
import functools

import jax
import jax.numpy as jnp
from jax import lax
from jax.experimental import pallas as pl
from jax.experimental.pallas import tpu as pltpu
from jax.experimental.pallas import tpu_sc as plsc

NC = 2
NS = 16
NW = NC * NS
CHUNK = 40
SEG = 50


def _round_up(v, m):
    return (v + m - 1) // m * m


def _agg_body(pipe, n, d, e, *refs):
    (tabb, srcb, dstb, tabc, srcc, dstc,
     tabu, srcu, dstu, pmu, tabd, srcd, dstd, pmd, zblk,
     outb, outc, outu, outd, acc, sbuf, didx_seg, rows, pmv,
     *sems) = refs
    gsems, psems, ssems, isems = (sems[:pipe], sems[pipe:2 * pipe],
                                  sems[2 * pipe:3 * pipe],
                                  sems[3 * pipe:])
    cid = lax.axis_index("c")
    sid = lax.axis_index("s")
    wid = cid * NS + sid
    epw = e // NW
    nchunks = epw // CHUNK
    ngroups = nchunks // pipe
    npad = _round_up(n, 8 * NS)
    rps = npad // NS
    rslice = pl.ds(sid * rps, rps)

    ngseg = SEG // pipe
    nseg = nchunks // SEG

    def one_adjacency(tab, src, dst, pm, out):
        pltpu.sync_copy(zblk, acc.at[rslice])
        plsc.subcore_barrier()

        def sidx_desc(c, b):
            return pltpu.make_async_copy(
                src.at[pl.ds(wid * epw + c * CHUNK, CHUNK)], sbuf.at[b],
                isems[b])

        def gather_desc(b):
            return pltpu.make_async_copy(tab.at[sbuf.at[b]], rows.at[b],
                                         gsems[b])

        def pm_desc(c, b):
            return pltpu.make_async_copy(pm.at[wid * nchunks + c],
                                         pmv.at[b], psems[b])

        def scatter_desc(cl, b):
            return pltpu.make_async_copy(
                rows.at[b], acc.at[didx_seg.at[cl]], ssems[b])

        for b in range(pipe):
            sidx_desc(b, b).start()

        @pl.loop(0, nseg)
        def _segs(s):
            @pl.when(s > 0)
            def _():
                for b in range(pipe):
                    scatter_desc(SEG - pipe + b, b).wait()
            pltpu.sync_copy(dst.at[wid * nseg + s], didx_seg)

            @pl.loop(0, ngseg)
            def _groups(o):
                for b in range(pipe):
                    cl = o * pipe + b

                    @pl.when(o > 0)
                    def _():
                        scatter_desc(cl - pipe, b).wait()
                    sidx_desc(s * SEG + cl, b).wait()
                    if pm is not None:
                        pm_desc(s * SEG + cl, b).start()
                    gather_desc(b).start()
                for b in range(pipe):
                    cl = o * pipe + b
                    c = s * SEG + cl
                    gather_desc(b).wait()

                    @pl.when(c + pipe < nchunks)
                    def _():
                        sidx_desc(c + pipe, b).start()
                    if pm is not None:
                        pm_desc(c, b).wait()

                        def unpack_rows(r0ofs, r1ofs):
                            def body(rr):
                                for j in range(d // 16):
                                    sl = pl.ds(16 * j, 16)
                                    v = pmv[b, rr, sl]
                                    pa = jax.lax.bitcast_convert_type(
                                        jax.lax.shift_left(v, 16),
                                        jnp.float32)
                                    pb = jax.lax.bitcast_convert_type(
                                        v & jnp.int32(-65536), jnp.float32)
                                    ra = rr + r0ofs
                                    rb = rr + r1ofs
                                    rows[b, ra, sl] = jnp.maximum(
                                        rows[b, ra, sl] + pa, 0.0)
                                    rows[b, rb, sl] = jnp.maximum(
                                        rows[b, rb, sl] + pb, 0.0)
                            return body
                        plsc.parallel_loop(0, 8, unroll=2)(
                            unpack_rows(0, 8))
                        plsc.parallel_loop(8, 16, unroll=2)(
                            unpack_rows(8, 16))
                        plsc.parallel_loop(16, 20, unroll=2)(
                            unpack_rows(16, 20))
                    pltpu.async_copy(rows.at[b], acc.at[didx_seg.at[cl]],
                                     ssems[b], add=True)

        for b in range(pipe):
            scatter_desc(SEG - pipe + b, b).wait()
        plsc.subcore_barrier()
        pltpu.sync_copy(acc.at[rslice], out.at[cid, rslice])
        plsc.subcore_barrier()

    one_adjacency(tabb, srcb, dstb, None, outb)
    one_adjacency(tabc, srcc, dstc, None, outc)
    one_adjacency(tabu, srcu, dstu, pmu, outu)
    one_adjacency(tabd, srcd, dstd, pmd, outd)


def _make_agg(n, d, e):
    pipe = 5
    npad = _round_up(n, 8 * NS)
    scratch = [
        pltpu.VMEM_SHARED((npad, d), jnp.float32),
        pltpu.VMEM((pipe, CHUNK), jnp.int32),
        pltpu.VMEM((SEG, CHUNK), jnp.int32),
        pltpu.VMEM((pipe, CHUNK, d), jnp.float32),
        pltpu.VMEM((pipe, CHUNK // 2, d), jnp.int32),
    ]
    scratch.extend([pltpu.SemaphoreType.DMA] * (4 * pipe))
    out = (jax.ShapeDtypeStruct((NC, npad, d), jnp.float32),) * 4
    return pl.kernel(
        functools.partial(_agg_body, pipe, n, d, e),
        out_type=out,
        mesh=plsc.VectorSubcoreMesh(core_axis_name="c", subcore_axis_name="s"),
        scratch_types=scratch,
        name="sc_agg_all",
    )


def _prep_x_body(x_ref, w0_ref, w1_ref, o0_ref, o1_ref):
    x = x_ref[...]
    o0_ref[...] = jnp.dot(x, w0_ref[...], preferred_element_type=jnp.float32)
    o1_ref[...] = jnp.dot(x, w1_ref[...], preferred_element_type=jnp.float32)


def _pack_words(p, nc, ck2, d):
    m = jnp.uint32(0xFFFF0000)
    g = jax.lax.bitcast_convert_type(p, jnp.uint32).reshape(nc, 5, 8, d)
    w01 = (g[:, 0] >> 16) | (g[:, 1] & m)
    w23 = (g[:, 2] >> 16) | (g[:, 3] & m)
    w4 = (g[:, 4, :4] >> 16) | (g[:, 4, 4:] & m)
    return jax.lax.bitcast_convert_type(
        jnp.concatenate([w01, w23, w4], axis=1), jnp.int32)


def _prep_attr_body(a0_ref, w0_ref, b0_ref, a1_ref, w1_ref, b1_ref,
                    o0_ref, o1_ref):
    nc, ck2, d = o0_ref.shape
    p0 = (jnp.dot(a0_ref[...], w0_ref[...],
                  preferred_element_type=jnp.float32) + b0_ref[...])
    o0_ref[...] = _pack_words(p0, nc, ck2, d)
    p1 = (jnp.dot(a1_ref[...], w1_ref[...],
                  preferred_element_type=jnp.float32) + b1_ref[...])
    o1_ref[...] = _pack_words(p1, nc, ck2, d)


def _final_body(x_ref, pu_ref, pd_ref, pb_ref, pc_ref,
                wu_ref, bu_ref, wd_ref, bd_ref, wb_ref, bb_ref,
                wc_ref, bc_ref, wcomb_ref, bcomb_ref, o_ref):
    x = x_ref[...]
    acc = bcomb_ref[...].astype(jnp.float32)
    for k, (p_ref, w_ref, b_ref) in enumerate((
            (pu_ref, wu_ref, bu_ref), (pd_ref, wd_ref, bd_ref),
            (pb_ref, wb_ref, bb_ref), (pc_ref, wc_ref, bc_ref))):
        agg = p_ref[0] + p_ref[1] + x
        h = jnp.maximum(
            jnp.dot(agg, w_ref[...], preferred_element_type=jnp.float32)
            + b_ref[...], 0.0)
        acc = acc + jnp.dot(h, wcomb_ref[k], preferred_element_type=jnp.float32)
    o_ref[...] = jnp.maximum(acc, 0.0)


def kernel(x, up_index, down_index, boundary_index, coboundary_index,
           up_attr, down_attr,
           W_msg_up, b_msg_up, W_msg_down, b_msg_down,
           W_up, b_up, W_down, b_down, W_bnd, b_bnd, W_cob, b_cob,
           W_comb, b_comb):
    n, d = x.shape
    e = up_attr.shape[0]
    f32 = jnp.float32

    xtabs = pl.pallas_call(
        _prep_x_body,
        out_shape=(jax.ShapeDtypeStruct((n, d), f32),) * 2,
        name="tc_prep_x",
    )(x, W_msg_up[:d], W_msg_down[:d])
    xu, xd = xtabs

    be = 2560
    grid = e // be
    blk = pl.BlockSpec((be, d), lambda i: (i, 0))
    oblk = pl.BlockSpec((be // CHUNK, CHUNK // 2, d), lambda i: (i, 0, 0))
    wspec = pl.BlockSpec((d, d), lambda i: (0, 0))
    bspec = pl.BlockSpec((1, d), lambda i: (0, 0))
    pm_up, pm_down = pl.pallas_call(
        _prep_attr_body,
        grid=(grid,),
        in_specs=[blk, wspec, bspec, blk, wspec, bspec],
        out_specs=(oblk, oblk),
        out_shape=(jax.ShapeDtypeStruct((e // CHUNK, CHUNK // 2, d),
                                        jnp.int32),) * 2,
        name="tc_prep_attr",
    )(up_attr, W_msg_up[d:], b_msg_up.reshape(1, d),
      down_attr, W_msg_down[d:], b_msg_down.reshape(1, d))

    zblk = jnp.zeros((_round_up(n, 8 * NS) // NS, d), f32)
    epw = e // NW
    nchunks = epw // CHUNK

    def _sidx(a):
        return a

    def _didx(a):
        return a.reshape(NW * (nchunks // SEG), SEG, CHUNK)

    pb, pc, pu, pd = _make_agg(n, d, e)(
        x, _sidx(boundary_index[0]), _didx(boundary_index[1]),
        x, _sidx(coboundary_index[0]), _didx(coboundary_index[1]),
        xu, _sidx(up_index[0]), _didx(up_index[1]), pm_up,
        xd, _sidx(down_index[0]), _didx(down_index[1]), pm_down, zblk)

    bn = 2000
    gridn = n // bn
    xspec = pl.BlockSpec((bn, d), lambda i: (i, 0))
    pspec = pl.BlockSpec((NC, bn, d), lambda i: (0, i, 0))
    wspec2 = pl.BlockSpec((d, d), lambda i: (0, 0))
    bspec2 = pl.BlockSpec((1, d), lambda i: (0, 0))
    out = pl.pallas_call(
        _final_body,
        grid=(gridn,),
        in_specs=[xspec, pspec, pspec, pspec, pspec,
                  wspec2, bspec2, wspec2, bspec2, wspec2, bspec2,
                  wspec2, bspec2,
                  pl.BlockSpec((4, d, d), lambda i: (0, 0, 0)), bspec2],
        out_specs=xspec,
        out_shape=jax.ShapeDtypeStruct((n, d), f32),
        name="tc_final",
    )(x, pu, pd, pb, pc,
      W_up, b_up.reshape(1, d), W_down, b_down.reshape(1, d),
      W_bnd, b_bnd.reshape(1, d), W_cob, b_cob.reshape(1, d),
      W_comb.reshape(4, d, d), b_comb.reshape(1, d))
    return out


# speedup vs baseline: 1.0179x; 1.0179x over previous
import functools

import jax
import jax.numpy as jnp
from jax import lax
from jax.experimental import pallas as pl
from jax.experimental.pallas import tpu as pltpu
from jax.experimental.pallas import tpu_sc as plsc

NC = 2
NS = 16
NW = NC * NS
CHUNK = 40
SEG = 50


def _round_up(v, m):
    return (v + m - 1) // m * m


def _agg_body(msg, pipe, n, d, e, *refs):
    if msg:
        (tab0, src0, dst0, pm0, tab1, src1, dst1, pm1, zblk,
         out0, out1, acc, sbuf, didx_seg, rows, pmv,
         *sems) = refs
    else:
        (tab0, src0, dst0, tab1, src1, dst1, zblk,
         out0, out1, acc, sbuf, didx_seg, rows, *sems) = refs
        pm0 = pm1 = pmv = None
    gsems, psems, ssems, isems = (sems[:pipe], sems[pipe:2 * pipe],
                                  sems[2 * pipe:3 * pipe],
                                  sems[3 * pipe:])
    cid = lax.axis_index("c")
    sid = lax.axis_index("s")
    wid = cid * NS + sid
    epw = e // NW
    nchunks = epw // CHUNK
    ngroups = nchunks // pipe
    npad = _round_up(n, 8 * NS)
    rps = npad // NS
    rslice = pl.ds(sid * rps, rps)

    ngseg = SEG // pipe
    nseg = nchunks // SEG

    def one_adjacency(tab, src, dst, pm, out):
        pltpu.sync_copy(zblk, acc.at[rslice])
        plsc.subcore_barrier()

        def sidx_desc(c, b):
            return pltpu.make_async_copy(
                src.at[pl.ds(wid * epw + c * CHUNK, CHUNK)], sbuf.at[b],
                isems[b])

        def gather_desc(b):
            return pltpu.make_async_copy(tab.at[sbuf.at[b]], rows.at[b],
                                         gsems[b])

        def pm_desc(c, b):
            return pltpu.make_async_copy(pm.at[wid * nchunks + c],
                                         pmv.at[b], psems[b])

        def scatter_desc(cl, b):
            return pltpu.make_async_copy(
                rows.at[b], acc.at[didx_seg.at[cl]], ssems[b])

        for b in range(pipe):
            sidx_desc(b, b).start()

        @pl.loop(0, nseg)
        def _segs(s):
            @pl.when(s > 0)
            def _():
                for b in range(pipe):
                    scatter_desc(SEG - pipe + b, b).wait()
            pltpu.sync_copy(dst.at[wid * nseg + s], didx_seg)

            @pl.loop(0, ngseg)
            def _groups(o):
                for b in range(pipe):
                    cl = o * pipe + b

                    @pl.when(o > 0)
                    def _():
                        scatter_desc(cl - pipe, b).wait()
                    sidx_desc(s * SEG + cl, b).wait()
                    if pm is not None:
                        pm_desc(s * SEG + cl, b).start()
                    gather_desc(b).start()
                for b in range(pipe):
                    cl = o * pipe + b
                    c = s * SEG + cl
                    gather_desc(b).wait()

                    @pl.when(c + pipe < nchunks)
                    def _():
                        sidx_desc(c + pipe, b).start()
                    if pm is not None:
                        pm_desc(c, b).wait()

                        def unpack_rows(r0ofs, r1ofs):
                            def body(rr):
                                for j in range(d // 16):
                                    sl = pl.ds(16 * j, 16)
                                    v = pmv[b, rr, sl]
                                    pa = jax.lax.bitcast_convert_type(
                                        jax.lax.shift_left(v, 16),
                                        jnp.float32)
                                    pb = jax.lax.bitcast_convert_type(
                                        v & jnp.int32(-65536), jnp.float32)
                                    ra = rr + r0ofs
                                    rb = rr + r1ofs
                                    rows[b, ra, sl] = jnp.maximum(
                                        rows[b, ra, sl] + pa, 0.0)
                                    rows[b, rb, sl] = jnp.maximum(
                                        rows[b, rb, sl] + pb, 0.0)
                            return body
                        plsc.parallel_loop(0, 8, unroll=2)(
                            unpack_rows(0, 8))
                        plsc.parallel_loop(8, 16, unroll=2)(
                            unpack_rows(8, 16))
                        plsc.parallel_loop(16, 20, unroll=2)(
                            unpack_rows(16, 20))
                    pltpu.async_copy(rows.at[b], acc.at[didx_seg.at[cl]],
                                     ssems[b], add=True)

        for b in range(pipe):
            scatter_desc(SEG - pipe + b, b).wait()
        plsc.subcore_barrier()
        pltpu.sync_copy(acc.at[rslice], out.at[cid, rslice])
        plsc.subcore_barrier()

    one_adjacency(tab0, src0, dst0, pm0, out0)
    one_adjacency(tab1, src1, dst1, pm1, out1)


def _make_agg(msg, n, d, e):
    pipe = 5
    npad = _round_up(n, 8 * NS)
    scratch = [
        pltpu.VMEM_SHARED((npad, d), jnp.float32),
        pltpu.VMEM((pipe, CHUNK), jnp.int32),
        pltpu.VMEM((SEG, CHUNK), jnp.int32),
        pltpu.VMEM((pipe, CHUNK, d), jnp.float32),
    ]
    if msg:
        scratch.append(pltpu.VMEM((pipe, CHUNK // 2, d), jnp.int32))
    scratch.extend([pltpu.SemaphoreType.DMA] * (4 * pipe))
    out = (jax.ShapeDtypeStruct((NC, npad, d), jnp.float32),) * 2
    return pl.kernel(
        functools.partial(_agg_body, msg, pipe, n, d, e),
        out_type=out,
        mesh=plsc.VectorSubcoreMesh(core_axis_name="c", subcore_axis_name="s"),
        scratch_types=scratch,
        name="sc_agg_msg" if msg else "sc_agg_plain",
    )


def _prep_x_body(x_ref, w0_ref, w1_ref, o0_ref, o1_ref):
    x = x_ref[...]
    o0_ref[...] = jnp.dot(x, w0_ref[...], preferred_element_type=jnp.float32)
    o1_ref[...] = jnp.dot(x, w1_ref[...], preferred_element_type=jnp.float32)


def _pack_words(p, nc, ck2, d):
    m = jnp.uint32(0xFFFF0000)
    g = jax.lax.bitcast_convert_type(p, jnp.uint32).reshape(nc, 5, 8, d)
    w01 = (g[:, 0] >> 16) | (g[:, 1] & m)
    w23 = (g[:, 2] >> 16) | (g[:, 3] & m)
    w4 = (g[:, 4, :4] >> 16) | (g[:, 4, 4:] & m)
    return jax.lax.bitcast_convert_type(
        jnp.concatenate([w01, w23, w4], axis=1), jnp.int32)


def _prep_attr_body(a0_ref, w0_ref, b0_ref, a1_ref, w1_ref, b1_ref,
                    o0_ref, o1_ref):
    nc, ck2, d = o0_ref.shape
    p0 = (jnp.dot(a0_ref[...], w0_ref[...],
                  preferred_element_type=jnp.float32) + b0_ref[...])
    o0_ref[...] = _pack_words(p0, nc, ck2, d)
    p1 = (jnp.dot(a1_ref[...], w1_ref[...],
                  preferred_element_type=jnp.float32) + b1_ref[...])
    o1_ref[...] = _pack_words(p1, nc, ck2, d)


def _final_body(x_ref, pu_ref, pd_ref, pb_ref, pc_ref,
                wu_ref, bu_ref, wd_ref, bd_ref, wb_ref, bb_ref,
                wc_ref, bc_ref, wcomb_ref, bcomb_ref, o_ref):
    x = x_ref[...]
    acc = bcomb_ref[...].astype(jnp.float32)
    for k, (p_ref, w_ref, b_ref) in enumerate((
            (pu_ref, wu_ref, bu_ref), (pd_ref, wd_ref, bd_ref),
            (pb_ref, wb_ref, bb_ref), (pc_ref, wc_ref, bc_ref))):
        agg = p_ref[0] + p_ref[1] + x
        h = jnp.maximum(
            jnp.dot(agg, w_ref[...], preferred_element_type=jnp.float32)
            + b_ref[...], 0.0)
        acc = acc + jnp.dot(h, wcomb_ref[k], preferred_element_type=jnp.float32)
    o_ref[...] = jnp.maximum(acc, 0.0)


def kernel(x, up_index, down_index, boundary_index, coboundary_index,
           up_attr, down_attr,
           W_msg_up, b_msg_up, W_msg_down, b_msg_down,
           W_up, b_up, W_down, b_down, W_bnd, b_bnd, W_cob, b_cob,
           W_comb, b_comb):
    n, d = x.shape
    e = up_attr.shape[0]
    f32 = jnp.float32

    xtabs = pl.pallas_call(
        _prep_x_body,
        out_shape=(jax.ShapeDtypeStruct((n, d), f32),) * 2,
        name="tc_prep_x",
    )(x, W_msg_up[:d], W_msg_down[:d])
    xu, xd = xtabs

    be = 2560
    grid = e // be
    blk = pl.BlockSpec((be, d), lambda i: (i, 0))
    oblk = pl.BlockSpec((be // CHUNK, CHUNK // 2, d), lambda i: (i, 0, 0))
    wspec = pl.BlockSpec((d, d), lambda i: (0, 0))
    bspec = pl.BlockSpec((1, d), lambda i: (0, 0))
    pm_up, pm_down = pl.pallas_call(
        _prep_attr_body,
        grid=(grid,),
        in_specs=[blk, wspec, bspec, blk, wspec, bspec],
        out_specs=(oblk, oblk),
        out_shape=(jax.ShapeDtypeStruct((e // CHUNK, CHUNK // 2, d),
                                        jnp.int32),) * 2,
        name="tc_prep_attr",
    )(up_attr, W_msg_up[d:], b_msg_up.reshape(1, d),
      down_attr, W_msg_down[d:], b_msg_down.reshape(1, d))

    zblk = jnp.zeros((_round_up(n, 8 * NS) // NS, d), f32)
    epw = e // NW
    nchunks = epw // CHUNK

    def _sidx(a):
        return a

    def _didx(a):
        return a.reshape(NW * (nchunks // SEG), SEG, CHUNK)

    pb, pc = _make_agg(False, n, d, e)(
        x, _sidx(boundary_index[0]), _didx(boundary_index[1]),
        x, _sidx(coboundary_index[0]), _didx(coboundary_index[1]), zblk)

    pu, pd = _make_agg(True, n, d, e)(
        xu, _sidx(up_index[0]), _didx(up_index[1]), pm_up,
        xd, _sidx(down_index[0]), _didx(down_index[1]), pm_down, zblk)

    bn = 2000
    gridn = n // bn
    xspec = pl.BlockSpec((bn, d), lambda i: (i, 0))
    pspec = pl.BlockSpec((NC, bn, d), lambda i: (0, i, 0))
    wspec2 = pl.BlockSpec((d, d), lambda i: (0, 0))
    bspec2 = pl.BlockSpec((1, d), lambda i: (0, 0))
    out = pl.pallas_call(
        _final_body,
        grid=(gridn,),
        in_specs=[xspec, pspec, pspec, pspec, pspec,
                  wspec2, bspec2, wspec2, bspec2, wspec2, bspec2,
                  wspec2, bspec2,
                  pl.BlockSpec((4, d, d), lambda i: (0, 0, 0)), bspec2],
        out_specs=xspec,
        out_shape=jax.ShapeDtypeStruct((n, d), f32),
        name="tc_final",
    )(x, pu, pd, pb, pc,
      W_up, b_up.reshape(1, d), W_down, b_down.reshape(1, d),
      W_bnd, b_bnd.reshape(1, d), W_cob, b_cob.reshape(1, d),
      W_comb.reshape(4, d, d), b_comb.reshape(1, d))
    return out


# speedup vs baseline: 1.2338x; 1.2121x over previous
import functools

import jax
import jax.numpy as jnp
from jax import lax
from jax.experimental import pallas as pl
from jax.experimental.pallas import tpu as pltpu
from jax.experimental.pallas import tpu_sc as plsc

NC = 2
NS = 16
NW = NC * NS
CHUNK = 40
SEG = 50


def _round_up(v, m):
    return (v + m - 1) // m * m


def _agg_body(msg, pipe, n, d, e, *refs):
    if msg:
        (tab0, src0, dst0, pm0, tab1, src1, dst1, pm1, zblk,
         out0, out1, acc, sbuf, didx_seg, rows, pmv,
         *sems) = refs
    else:
        (tab0, src0, dst0, tab1, src1, dst1, zblk,
         out0, out1, acc, sbuf, didx_seg, rows, *sems) = refs
        pm0 = pm1 = pmv = None
    gsems, psems, ssems, isems = (sems[:pipe], sems[pipe:2 * pipe],
                                  sems[2 * pipe:3 * pipe],
                                  sems[3 * pipe:])
    cid = lax.axis_index("c")
    sid = lax.axis_index("s")
    wid = cid * NS + sid
    epw = e // NW
    nchunks = epw // CHUNK
    ngroups = nchunks // pipe
    npad = _round_up(n, 8 * NS)
    rps = npad // NS
    rslice = pl.ds(sid * rps, rps)

    ngseg = SEG // pipe
    nseg = nchunks // SEG

    def one_adjacency(tab, src, dst, pm, out):
        pltpu.sync_copy(zblk, acc.at[rslice])
        plsc.subcore_barrier()

        def sidx_desc(c, b):
            return pltpu.make_async_copy(
                src.at[pl.ds(wid * epw + c * CHUNK, CHUNK)], sbuf.at[b],
                isems[b])

        def gather_desc(b):
            return pltpu.make_async_copy(tab.at[sbuf.at[b]], rows.at[b],
                                         gsems[b])

        def pm_desc(c, b):
            return pltpu.make_async_copy(pm.at[wid * nchunks + c],
                                         pmv.at[b], psems[b])

        def scatter_desc(cl, b):
            return pltpu.make_async_copy(
                rows.at[b], acc.at[didx_seg.at[cl]], ssems[b])

        for b in range(pipe):
            sidx_desc(b, b).start()

        @pl.loop(0, nseg)
        def _segs(s):
            @pl.when(s > 0)
            def _():
                for b in range(pipe):
                    scatter_desc(SEG - pipe + b, b).wait()
            pltpu.sync_copy(dst.at[wid * nseg + s], didx_seg)

            @pl.loop(0, ngseg)
            def _groups(o):
                for b in range(pipe):
                    cl = o * pipe + b

                    @pl.when(o > 0)
                    def _():
                        scatter_desc(cl - pipe, b).wait()
                    sidx_desc(s * SEG + cl, b).wait()
                    if pm is not None:
                        pm_desc(s * SEG + cl, b).start()
                    gather_desc(b).start()
                for b in range(pipe):
                    cl = o * pipe + b
                    c = s * SEG + cl
                    gather_desc(b).wait()

                    @pl.when(c + pipe < nchunks)
                    def _():
                        sidx_desc(c + pipe, b).start()
                    if pm is not None:
                        pm_desc(c, b).wait()

                        @plsc.parallel_loop(0, CHUNK // 2, unroll=2)
                        def _rows(rr):
                            ra = rr + 8 * jnp.minimum(rr // 8, 2)
                            rb = ra + jnp.where(rr < 16, 8, 4)
                            for j in range(d // 16):
                                sl = pl.ds(16 * j, 16)
                                v = pmv[b, rr, sl]
                                pa = jax.lax.bitcast_convert_type(
                                    jax.lax.shift_left(v, 16), jnp.float32)
                                pb = jax.lax.bitcast_convert_type(
                                    v & jnp.int32(-65536), jnp.float32)
                                rows[b, ra, sl] = jnp.maximum(
                                    rows[b, ra, sl] + pa, 0.0)
                                rows[b, rb, sl] = jnp.maximum(
                                    rows[b, rb, sl] + pb, 0.0)
                    pltpu.async_copy(rows.at[b], acc.at[didx_seg.at[cl]],
                                     ssems[b], add=True)

        for b in range(pipe):
            scatter_desc(SEG - pipe + b, b).wait()
        plsc.subcore_barrier()
        pltpu.sync_copy(acc.at[rslice], out.at[cid, rslice])
        plsc.subcore_barrier()

    one_adjacency(tab0, src0, dst0, pm0, out0)
    one_adjacency(tab1, src1, dst1, pm1, out1)


def _make_agg(msg, n, d, e):
    pipe = 5
    npad = _round_up(n, 8 * NS)
    scratch = [
        pltpu.VMEM_SHARED((npad, d), jnp.float32),
        pltpu.VMEM((pipe, CHUNK), jnp.int32),
        pltpu.VMEM((SEG, CHUNK), jnp.int32),
        pltpu.VMEM((pipe, CHUNK, d), jnp.float32),
    ]
    if msg:
        scratch.append(pltpu.VMEM((pipe, CHUNK // 2, d), jnp.int32))
    scratch.extend([pltpu.SemaphoreType.DMA] * (4 * pipe))
    out = (jax.ShapeDtypeStruct((NC, npad, d), jnp.float32),) * 2
    return pl.kernel(
        functools.partial(_agg_body, msg, pipe, n, d, e),
        out_type=out,
        mesh=plsc.VectorSubcoreMesh(core_axis_name="c", subcore_axis_name="s"),
        scratch_types=scratch,
        name="sc_agg_msg" if msg else "sc_agg_plain",
    )


def _prep_x_body(x_ref, w0_ref, w1_ref, o0_ref, o1_ref):
    x = x_ref[...]
    o0_ref[...] = jnp.dot(x, w0_ref[...], preferred_element_type=jnp.float32)
    o1_ref[...] = jnp.dot(x, w1_ref[...], preferred_element_type=jnp.float32)


def _pack_words(p, nc, ck2, d):
    m = jnp.uint32(0xFFFF0000)
    g = jax.lax.bitcast_convert_type(p, jnp.uint32).reshape(nc, 5, 8, d)
    w01 = (g[:, 0] >> 16) | (g[:, 1] & m)
    w23 = (g[:, 2] >> 16) | (g[:, 3] & m)
    w4 = (g[:, 4, :4] >> 16) | (g[:, 4, 4:] & m)
    return jax.lax.bitcast_convert_type(
        jnp.concatenate([w01, w23, w4], axis=1), jnp.int32)


def _prep_attr_body(a0_ref, w0_ref, b0_ref, a1_ref, w1_ref, b1_ref,
                    o0_ref, o1_ref):
    nc, ck2, d = o0_ref.shape
    p0 = (jnp.dot(a0_ref[...], w0_ref[...],
                  preferred_element_type=jnp.float32) + b0_ref[...])
    o0_ref[...] = _pack_words(p0, nc, ck2, d)
    p1 = (jnp.dot(a1_ref[...], w1_ref[...],
                  preferred_element_type=jnp.float32) + b1_ref[...])
    o1_ref[...] = _pack_words(p1, nc, ck2, d)


def _final_body(x_ref, pu_ref, pd_ref, pb_ref, pc_ref,
                wu_ref, bu_ref, wd_ref, bd_ref, wb_ref, bb_ref,
                wc_ref, bc_ref, wcomb_ref, bcomb_ref, o_ref):
    x = x_ref[...]
    acc = bcomb_ref[...].astype(jnp.float32)
    for k, (p_ref, w_ref, b_ref) in enumerate((
            (pu_ref, wu_ref, bu_ref), (pd_ref, wd_ref, bd_ref),
            (pb_ref, wb_ref, bb_ref), (pc_ref, wc_ref, bc_ref))):
        agg = p_ref[0] + p_ref[1] + x
        h = jnp.maximum(
            jnp.dot(agg, w_ref[...], preferred_element_type=jnp.float32)
            + b_ref[...], 0.0)
        acc = acc + jnp.dot(h, wcomb_ref[k], preferred_element_type=jnp.float32)
    o_ref[...] = jnp.maximum(acc, 0.0)


def kernel(x, up_index, down_index, boundary_index, coboundary_index,
           up_attr, down_attr,
           W_msg_up, b_msg_up, W_msg_down, b_msg_down,
           W_up, b_up, W_down, b_down, W_bnd, b_bnd, W_cob, b_cob,
           W_comb, b_comb):
    n, d = x.shape
    e = up_attr.shape[0]
    f32 = jnp.float32

    xtabs = pl.pallas_call(
        _prep_x_body,
        out_shape=(jax.ShapeDtypeStruct((n, d), f32),) * 2,
        name="tc_prep_x",
    )(x, W_msg_up[:d], W_msg_down[:d])
    xu, xd = xtabs

    be = 2560
    grid = e // be
    blk = pl.BlockSpec((be, d), lambda i: (i, 0))
    oblk = pl.BlockSpec((be // CHUNK, CHUNK // 2, d), lambda i: (i, 0, 0))
    wspec = pl.BlockSpec((d, d), lambda i: (0, 0))
    bspec = pl.BlockSpec((1, d), lambda i: (0, 0))
    pm_up, pm_down = pl.pallas_call(
        _prep_attr_body,
        grid=(grid,),
        in_specs=[blk, wspec, bspec, blk, wspec, bspec],
        out_specs=(oblk, oblk),
        out_shape=(jax.ShapeDtypeStruct((e // CHUNK, CHUNK // 2, d),
                                        jnp.int32),) * 2,
        name="tc_prep_attr",
    )(up_attr, W_msg_up[d:], b_msg_up.reshape(1, d),
      down_attr, W_msg_down[d:], b_msg_down.reshape(1, d))

    zblk = jnp.zeros((_round_up(n, 8 * NS) // NS, d), f32)
    epw = e // NW
    nchunks = epw // CHUNK

    def _sidx(a):
        return a

    def _didx(a):
        return a.reshape(NW * (nchunks // SEG), SEG, CHUNK)

    pb, pc = _make_agg(False, n, d, e)(
        x, _sidx(boundary_index[0]), _didx(boundary_index[1]),
        x, _sidx(coboundary_index[0]), _didx(coboundary_index[1]), zblk)

    pu, pd = _make_agg(True, n, d, e)(
        xu, _sidx(up_index[0]), _didx(up_index[1]), pm_up,
        xd, _sidx(down_index[0]), _didx(down_index[1]), pm_down, zblk)

    bn = 2000
    gridn = n // bn
    xspec = pl.BlockSpec((bn, d), lambda i: (i, 0))
    pspec = pl.BlockSpec((NC, bn, d), lambda i: (0, i, 0))
    wspec2 = pl.BlockSpec((d, d), lambda i: (0, 0))
    bspec2 = pl.BlockSpec((1, d), lambda i: (0, 0))
    out = pl.pallas_call(
        _final_body,
        grid=(gridn,),
        in_specs=[xspec, pspec, pspec, pspec, pspec,
                  wspec2, bspec2, wspec2, bspec2, wspec2, bspec2,
                  wspec2, bspec2,
                  pl.BlockSpec((4, d, d), lambda i: (0, 0, 0)), bspec2],
        out_specs=xspec,
        out_shape=jax.ShapeDtypeStruct((n, d), f32),
        name="tc_final",
    )(x, pu, pd, pb, pc,
      W_up, b_up.reshape(1, d), W_down, b_down.reshape(1, d),
      W_bnd, b_bnd.reshape(1, d), W_cob, b_cob.reshape(1, d),
      W_comb.reshape(4, d, d), b_comb.reshape(1, d))
    return out


# speedup vs baseline: 1.2357x; 1.0015x over previous
import functools

import jax
import jax.numpy as jnp
from jax import lax
from jax.experimental import pallas as pl
from jax.experimental.pallas import tpu as pltpu
from jax.experimental.pallas import tpu_sc as plsc

NC = 2
NS = 16
NW = NC * NS
CHUNK = 40
SEG = 50


def _round_up(v, m):
    return (v + m - 1) // m * m


def _agg_body(msg, pipe, n, d, e, *refs):
    if msg:
        (tab0, src0, dst0, pm0, tab1, src1, dst1, pm1, zblk,
         out0, out1, acc, sbuf, didx_seg, rows, pmv,
         *sems) = refs
    else:
        (tab0, src0, dst0, tab1, src1, dst1, zblk,
         out0, out1, acc, sbuf, didx_seg, rows, *sems) = refs
        pm0 = pm1 = pmv = None
    gsems, psems, ssems, isems = (sems[:pipe], sems[pipe:2 * pipe],
                                  sems[2 * pipe:3 * pipe],
                                  sems[3 * pipe:])
    cid = lax.axis_index("c")
    sid = lax.axis_index("s")
    wid = cid * NS + sid
    epw = e // NW
    nchunks = epw // CHUNK
    ngroups = nchunks // pipe
    npad = _round_up(n, 8 * NS)
    rps = npad // NS
    rslice = pl.ds(sid * rps, rps)

    ngseg = SEG // pipe
    nseg = nchunks // SEG

    def one_adjacency(tab, src, dst, pm, out):
        pltpu.sync_copy(zblk, acc.at[rslice])
        plsc.subcore_barrier()

        def sidx_desc(c, b):
            return pltpu.make_async_copy(
                src.at[pl.ds(wid * epw + c * CHUNK, CHUNK)], sbuf.at[b],
                isems[b])

        def gather_desc(b):
            return pltpu.make_async_copy(tab.at[sbuf.at[b]], rows.at[b],
                                         gsems[b])

        def pm_desc(c, b):
            return pltpu.make_async_copy(pm.at[wid * nchunks + c],
                                         pmv.at[b], psems[b])

        def scatter_desc(cl, b):
            return pltpu.make_async_copy(
                rows.at[b], acc.at[didx_seg.at[cl]], ssems[b])

        for b in range(pipe):
            sidx_desc(b, b).start()

        @pl.loop(0, nseg)
        def _segs(s):
            @pl.when(s > 0)
            def _():
                for b in range(pipe):
                    scatter_desc(SEG - pipe + b, b).wait()
            pltpu.sync_copy(dst.at[wid * nseg + s], didx_seg)

            @pl.loop(0, ngseg)
            def _groups(o):
                for b in range(pipe):
                    cl = o * pipe + b

                    @pl.when(o > 0)
                    def _():
                        scatter_desc(cl - pipe, b).wait()
                    sidx_desc(s * SEG + cl, b).wait()
                    if pm is not None:
                        pm_desc(s * SEG + cl, b).start()
                    gather_desc(b).start()
                for b in range(pipe):
                    cl = o * pipe + b
                    c = s * SEG + cl
                    gather_desc(b).wait()

                    @pl.when(c + pipe < nchunks)
                    def _():
                        sidx_desc(c + pipe, b).start()
                    if pm is not None:
                        pm_desc(c, b).wait()

                        @plsc.parallel_loop(0, CHUNK // 2, unroll=2)
                        def _rows(rr):
                            ra = rr + 8 * jnp.minimum(rr // 8, 2)
                            rb = ra + jnp.where(rr < 16, 8, 4)
                            for j in range(d // 16):
                                sl = pl.ds(16 * j, 16)
                                v = pmv[b, rr, sl]
                                pa = jax.lax.bitcast_convert_type(
                                    jax.lax.shift_left(v, 16), jnp.float32)
                                pb = jax.lax.bitcast_convert_type(
                                    v & jnp.int32(-65536), jnp.float32)
                                rows[b, ra, sl] = jnp.maximum(
                                    rows[b, ra, sl] + pa, 0.0)
                                rows[b, rb, sl] = jnp.maximum(
                                    rows[b, rb, sl] + pb, 0.0)
                    pltpu.async_copy(rows.at[b], acc.at[didx_seg.at[cl]],
                                     ssems[b], add=True)

        for b in range(pipe):
            scatter_desc(SEG - pipe + b, b).wait()
        plsc.subcore_barrier()
        pltpu.sync_copy(acc.at[rslice], out.at[cid, rslice])
        plsc.subcore_barrier()

    one_adjacency(tab0, src0, dst0, pm0, out0)
    one_adjacency(tab1, src1, dst1, pm1, out1)


def _make_agg(msg, n, d, e):
    pipe = 5
    npad = _round_up(n, 8 * NS)
    scratch = [
        pltpu.VMEM_SHARED((npad, d), jnp.float32),
        pltpu.VMEM((pipe, CHUNK), jnp.int32),
        pltpu.VMEM((SEG, CHUNK), jnp.int32),
        pltpu.VMEM((pipe, CHUNK, d), jnp.float32),
    ]
    if msg:
        scratch.append(pltpu.VMEM((pipe, CHUNK // 2, d), jnp.int32))
    scratch.extend([pltpu.SemaphoreType.DMA] * (4 * pipe))
    out = (jax.ShapeDtypeStruct((NC, npad, d), jnp.float32),) * 2
    return pl.kernel(
        functools.partial(_agg_body, msg, pipe, n, d, e),
        out_type=out,
        mesh=plsc.VectorSubcoreMesh(core_axis_name="c", subcore_axis_name="s"),
        scratch_types=scratch,
        name="sc_agg_msg" if msg else "sc_agg_plain",
    )


def _prep_x_body(x_ref, w0_ref, w1_ref, o0_ref, o1_ref):
    x = x_ref[...]
    o0_ref[...] = jnp.dot(x, w0_ref[...], preferred_element_type=jnp.float32)
    o1_ref[...] = jnp.dot(x, w1_ref[...], preferred_element_type=jnp.float32)


def _pack_words(p, nc, ck2, d):
    m = jnp.uint32(0xFFFF0000)
    g = jax.lax.bitcast_convert_type(p, jnp.uint32).reshape(nc, 5, 8, d)
    w01 = (g[:, 0] >> 16) | (g[:, 1] & m)
    w23 = (g[:, 2] >> 16) | (g[:, 3] & m)
    w4 = (g[:, 4, :4] >> 16) | (g[:, 4, 4:] & m)
    return jax.lax.bitcast_convert_type(
        jnp.concatenate([w01, w23, w4], axis=1), jnp.int32)


def _prep_attr_body(a0_ref, w0_ref, b0_ref, a1_ref, w1_ref, b1_ref,
                    x_ref, wx0_ref, wx1_ref,
                    o0_ref, o1_ref, xu_ref, xd_ref):
    nc, ck2, d = o0_ref.shape
    p0 = (jnp.dot(a0_ref[...], w0_ref[...],
                  preferred_element_type=jnp.float32) + b0_ref[...])
    o0_ref[...] = _pack_words(p0, nc, ck2, d)
    p1 = (jnp.dot(a1_ref[...], w1_ref[...],
                  preferred_element_type=jnp.float32) + b1_ref[...])
    o1_ref[...] = _pack_words(p1, nc, ck2, d)

    @pl.when(pl.program_id(0) == 0)
    def _():
        x = x_ref[...]
        xu_ref[...] = jnp.dot(x, wx0_ref[...],
                              preferred_element_type=jnp.float32)
        xd_ref[...] = jnp.dot(x, wx1_ref[...],
                              preferred_element_type=jnp.float32)


def _final_body(x_ref, pu_ref, pd_ref, pb_ref, pc_ref,
                wu_ref, bu_ref, wd_ref, bd_ref, wb_ref, bb_ref,
                wc_ref, bc_ref, wcomb_ref, bcomb_ref, o_ref):
    x = x_ref[...]
    acc = bcomb_ref[...].astype(jnp.float32)
    for k, (p_ref, w_ref, b_ref) in enumerate((
            (pu_ref, wu_ref, bu_ref), (pd_ref, wd_ref, bd_ref),
            (pb_ref, wb_ref, bb_ref), (pc_ref, wc_ref, bc_ref))):
        agg = p_ref[0] + p_ref[1] + x
        h = jnp.maximum(
            jnp.dot(agg, w_ref[...], preferred_element_type=jnp.float32)
            + b_ref[...], 0.0)
        acc = acc + jnp.dot(h, wcomb_ref[k], preferred_element_type=jnp.float32)
    o_ref[...] = jnp.maximum(acc, 0.0)


def kernel(x, up_index, down_index, boundary_index, coboundary_index,
           up_attr, down_attr,
           W_msg_up, b_msg_up, W_msg_down, b_msg_down,
           W_up, b_up, W_down, b_down, W_bnd, b_bnd, W_cob, b_cob,
           W_comb, b_comb):
    n, d = x.shape
    e = up_attr.shape[0]
    f32 = jnp.float32

    be = 2560
    grid = e // be
    blk = pl.BlockSpec((be, d), lambda i: (i, 0))
    oblk = pl.BlockSpec((be // CHUNK, CHUNK // 2, d), lambda i: (i, 0, 0))
    wspec = pl.BlockSpec((d, d), lambda i: (0, 0))
    bspec = pl.BlockSpec((1, d), lambda i: (0, 0))
    xspec0 = pl.BlockSpec((n, d), lambda i: (0, 0))
    pm_up, pm_down, xu, xd = pl.pallas_call(
        _prep_attr_body,
        grid=(grid,),
        in_specs=[blk, wspec, bspec, blk, wspec, bspec,
                  xspec0, wspec, wspec],
        out_specs=(oblk, oblk, xspec0, xspec0),
        out_shape=((jax.ShapeDtypeStruct((e // CHUNK, CHUNK // 2, d),
                                         jnp.int32),) * 2
                   + (jax.ShapeDtypeStruct((n, d), f32),) * 2),
        name="tc_prep_attr",
    )(up_attr, W_msg_up[d:], b_msg_up.reshape(1, d),
      down_attr, W_msg_down[d:], b_msg_down.reshape(1, d),
      x, W_msg_up[:d], W_msg_down[:d])

    zblk = jnp.zeros((_round_up(n, 8 * NS) // NS, d), f32)
    epw = e // NW
    nchunks = epw // CHUNK

    def _sidx(a):
        return a

    def _didx(a):
        return a.reshape(NW * (nchunks // SEG), SEG, CHUNK)

    pb, pc = _make_agg(False, n, d, e)(
        x, _sidx(boundary_index[0]), _didx(boundary_index[1]),
        x, _sidx(coboundary_index[0]), _didx(coboundary_index[1]), zblk)

    pu, pd = _make_agg(True, n, d, e)(
        xu, _sidx(up_index[0]), _didx(up_index[1]), pm_up,
        xd, _sidx(down_index[0]), _didx(down_index[1]), pm_down, zblk)

    bn = 2000
    gridn = n // bn
    xspec = pl.BlockSpec((bn, d), lambda i: (i, 0))
    pspec = pl.BlockSpec((NC, bn, d), lambda i: (0, i, 0))
    wspec2 = pl.BlockSpec((d, d), lambda i: (0, 0))
    bspec2 = pl.BlockSpec((1, d), lambda i: (0, 0))
    out = pl.pallas_call(
        _final_body,
        grid=(gridn,),
        in_specs=[xspec, pspec, pspec, pspec, pspec,
                  wspec2, bspec2, wspec2, bspec2, wspec2, bspec2,
                  wspec2, bspec2,
                  pl.BlockSpec((4, d, d), lambda i: (0, 0, 0)), bspec2],
        out_specs=xspec,
        out_shape=jax.ShapeDtypeStruct((n, d), f32),
        name="tc_final",
    )(x, pu, pd, pb, pc,
      W_up, b_up.reshape(1, d), W_down, b_down.reshape(1, d),
      W_bnd, b_bnd.reshape(1, d), W_cob, b_cob.reshape(1, d),
      W_comb.reshape(4, d, d), b_comb.reshape(1, d))
    return out
